# baseline (speedup 1.0000x reference)
import functools

import jax
import jax.numpy as jnp
from jax import lax
from jax.experimental import pallas as pl
from jax.experimental.pallas import tpu as pltpu
from jax.experimental.pallas import tpu_sc as plsc

B = 4096
NF = 13
NC = 26
V = 100000
D = 64
H1, H2, H3 = 512, 256, 128

R = NC * D
SLABS = R // 8
NWORK = 32
SLABS_PER_W = 7
FULL_TILES = V // 128
TAIL_W = V - FULL_TILES * 128
NT = 43
_CHUNKS = [(i * NT, NT) for i in range(FULL_TILES // NT)]
_REM = FULL_TILES - (FULL_TILES // NT) * NT
if _REM:
    _CHUNKS.append((FULL_TILES - _REM, _REM))
_CHUNKS.append((FULL_TILES, 1))
NVEC = B // 16


@functools.cache
def _make_sc_gather():
    mesh = plsc.VectorSubcoreMesh(core_axis_name="c", subcore_axis_name="s")

    @functools.partial(
        pl.kernel,
        mesh=mesh,
        compiler_params=pltpu.CompilerParams(needs_layout_passes=False),
        out_type=jax.ShapeDtypeStruct((R, B), jnp.float32),
        scratch_types=[
            pltpu.VMEM((B,), jnp.int32),
            pltpu.VMEM((NT, 8, 128), jnp.float32),
            pltpu.VMEM((NT, 8, 128), jnp.float32),
            pltpu.VMEM((8, B), jnp.float32),
            pltpu.VMEM((B + 32,), jnp.int32),
            pltpu.SemaphoreType.DMA,
            pltpu.SemaphoreType.DMA,
        ],
    )
    def _sc_gather(cat_hbm, tbl_hbm, tail_hbm, out_hbm,
                   idx_v, buf0, buf1, outb, hits, sem0, sem1):
        wid = lax.axis_index("s") * 2 + lax.axis_index("c")

        lane = lax.iota(jnp.int32, 16)
        svs = [jnp.full((16,), s, jnp.int32) for s in range(8)]
        bufs = (buf0, buf1)
        sems = (sem0, sem1)

        def issue_chunk(q, ci):
            k0, nt = _CHUNKS[ci]
            buf, sem = bufs[ci % 2], sems[ci % 2]
            if k0 >= FULL_TILES:
                pltpu.async_copy(tail_hbm.at[pl.ds(q * 8, 8)], buf.at[0], sem)
            else:
                def issue(kk, c):
                    pltpu.async_copy(
                        tbl_hbm.at[pl.ds(q * 8, 8),
                                   pl.ds((k0 + kk) * 128, 128)],
                        buf.at[kk], sem)
                    return c
                lax.fori_loop(0, nt, issue, 0)

        def wait_chunk(ci):
            k0, nt = _CHUNKS[ci]
            buf, sem = bufs[ci % 2], sems[ci % 2]
            if k0 >= FULL_TILES:
                pltpu.make_async_copy(tail_hbm.at[pl.ds(0, 8)],
                                      buf.at[0], sem).wait()
            else:
                def drain(kk, c):
                    pltpu.make_async_copy(
                        tbl_hbm.at[pl.ds(0, 8), pl.ds(0, 128)],
                        buf.at[kk], sem).wait()
                    return c
                lax.fori_loop(0, nt, drain, 0)

        def compact_chunk(k0, nt):
            c0 = k0 * 128

            def scan(g, cursor):
                ms, encs, prefs, cnts = [], [], [], []
                for u in range(8):
                    gg = g * 8 + u
                    iv = idx_v[pl.ds(gg * 16, 16)]
                    k = lax.shift_right_logical(iv, 7)
                    m = (k >= k0) & (k < k0 + nt)
                    ms.append(m)
                    encs.append(lax.shift_left(lane + gg * 16, 13) | (iv - c0))
                    prefs.append(plsc.cumsum(m.astype(jnp.int32)))
                    cnts.append(plsc.all_reduce_population_count(m))
                for u in range(8):
                    plsc.store_scatter(hits, [cursor + prefs[u] - 1], encs[u],
                                       mask=ms[u])
                    cursor = cursor + cnts[u]
                return cursor

            cursor = lax.fori_loop(0, NVEC // 8, scan,
                                   jnp.zeros((16,), jnp.int32))
            return jnp.max(cursor)

        def gather_chunk(ci, n):
            buf = bufs[ci % 2]
            nn = jnp.full((16,), 1, jnp.int32) * n

            def one(t, c):
                m2s, bs, kks, ls = [], [], [], []
                for u in range(2):
                    tt = t * 2 + u
                    e = hits[pl.ds(tt * 16, 16)]
                    m2 = (lane + tt * 16) < nn
                    vl = jnp.where(m2, e & 8191, 0)
                    m2s.append(m2)
                    bs.append(jnp.where(m2, lax.shift_right_logical(e, 13), 0))
                    kks.append(lax.shift_right_logical(vl, 7))
                    ls.append(vl & 127)
                gots = [[plsc.load_gather(buf, [kks[u], svs[s], ls[u]],
                                          mask=m2s[u]) for s in range(8)]
                        for u in range(2)]
                for u in range(2):
                    for s in range(8):
                        plsc.store_scatter(outb, [svs[s], bs[u]], gots[u][s],
                                           mask=m2s[u])
                return c
            lax.fori_loop(0, (n + 31) // 32, one, 0)

        def slab_body(j, carry):
            q = wid + NWORK * j

            @pl.when(q < SLABS)
            def _():
                f = q // 8
                pltpu.sync_copy(cat_hbm.at[pl.ds(f * B, B)], idx_v)
                issue_chunk(q, 0)
                for ci, (k0, nt) in enumerate(_CHUNKS):
                    if ci + 1 < len(_CHUNKS):
                        issue_chunk(q, ci + 1)
                    n = compact_chunk(k0, nt)
                    wait_chunk(ci)
                    gather_chunk(ci, n)
                pltpu.sync_copy(outb, out_hbm.at[pl.ds(q * 8, 8)])
            return carry

        lax.fori_loop(0, SLABS_PER_W, slab_body, 0)

    return _sc_gather


BB = 512


def _mlp_body(cont_ref, emb_ref, wc_ref, bc_ref, w1a_ref, w1b_ref, b1_ref,
              w2_ref, b2_ref, w3_ref, b3_ref, wo_ref, bo_ref, out_ref):
    x = jnp.dot(cont_ref[:], wc_ref[:], preferred_element_type=jnp.float32)
    x = x + bc_ref[:]
    a = jnp.dot(x, w1a_ref[:], preferred_element_type=jnp.float32)
    a = a + lax.dot_general(emb_ref[:], w1b_ref[:],
                            dimension_numbers=(((0,), (0,)), ((), ())),
                            preferred_element_type=jnp.float32)
    a = jnp.maximum(a + b1_ref[:], 0.0)
    a = jnp.maximum(jnp.dot(a, w2_ref[:], preferred_element_type=jnp.float32) + b2_ref[:], 0.0)
    a = jnp.maximum(jnp.dot(a, w3_ref[:], preferred_element_type=jnp.float32) + b3_ref[:], 0.0)
    o = jnp.dot(a, wo_ref[:], preferred_element_type=jnp.float32) + bo_ref[:]
    out_ref[:] = jax.nn.sigmoid(o)


def _mlp(cont, embT, W_cont, b_cont, W1a, W1b, b1, W2, b2, W3, b3, Wo, bo):
    grid = (B // BB,)
    full = lambda r, c: pl.BlockSpec((r, c), lambda i: (0, 0))
    return pl.pallas_call(
        _mlp_body,
        grid=grid,
        in_specs=[
            pl.BlockSpec((BB, NF), lambda i: (i, 0)),
            pl.BlockSpec((R, BB), lambda i: (0, i)),
            full(NF, D), full(1, D),
            full(D, H1), full(R, H1), full(1, H1),
            full(H1, H2), full(1, H2),
            full(H2, H3), full(1, H3),
            full(H3, 1), full(1, 1),
        ],
        out_specs=pl.BlockSpec((BB, 1), lambda i: (i, 0)),
        out_shape=jax.ShapeDtypeStruct((B, 1), jnp.float32),
    )(cont, embT, W_cont, b_cont, W1a, W1b, b1, W2, b2, W3, b3, Wo, bo)


def kernel(continuous_features, categorical_features, tables,
           W_cont, b_cont, W1, b1, W2, b2, W3, b3, Wo, bo):
    cat = categorical_features.astype(jnp.int32)
    cat_flat = cat.T.reshape(-1)
    tbl = tables.transpose(0, 2, 1).reshape(R, V)
    tail = jnp.pad(tbl[:, FULL_TILES * 128:], ((0, 0), (0, 128 - TAIL_W)))

    embT = _make_sc_gather()(cat_flat, tbl, tail)

    out = _mlp(continuous_features, embT,
               W_cont, b_cont.reshape(1, D),
               W1[:D], W1[D:], b1.reshape(1, H1),
               W2, b2.reshape(1, H2),
               W3, b3.reshape(1, H3),
               Wo, bo.reshape(1, 1))
    return out


# speedup vs baseline: 1.0798x; 1.0798x over previous
import functools

import jax
import jax.numpy as jnp
from jax import lax
from jax.experimental import pallas as pl
from jax.experimental.pallas import tpu as pltpu
from jax.experimental.pallas import tpu_sc as plsc

B = 4096
NF = 13
NC = 26
V = 100000
D = 64
H1, H2, H3 = 512, 256, 128

R = NC * D
SLABS = R // 8
NWORK = 32
SLABS_PER_W = 7
FULL_TILES = V // 128
TAIL_W = V - FULL_TILES * 128
NT = 43
_CHUNKS = [(i * NT, NT) for i in range(FULL_TILES // NT)]
_REM = FULL_TILES - (FULL_TILES // NT) * NT
if _REM:
    _CHUNKS.append((FULL_TILES - _REM, _REM))
_CHUNKS.append((FULL_TILES, 1))
NVEC = B // 16


@functools.cache
def _make_sc_gather():
    mesh = plsc.VectorSubcoreMesh(core_axis_name="c", subcore_axis_name="s")

    @functools.partial(
        pl.kernel,
        mesh=mesh,
        compiler_params=pltpu.CompilerParams(needs_layout_passes=False),
        out_type=jax.ShapeDtypeStruct((R, B), jnp.float32),
        scratch_types=[
            pltpu.VMEM((B,), jnp.int32),
            pltpu.VMEM((NT, 8, 128), jnp.float32),
            pltpu.VMEM((NT, 8, 128), jnp.float32),
            pltpu.VMEM((8, B), jnp.float32),
            pltpu.VMEM((B + 32,), jnp.int32),
            pltpu.SemaphoreType.DMA,
            pltpu.SemaphoreType.DMA,
        ],
    )
    def _sc_gather(cat_hbm, tbl_hbm, tail_hbm, out_hbm,
                   idx_v, buf0, buf1, outb, hits, sem0, sem1):
        wid = lax.axis_index("s") * 2 + lax.axis_index("c")

        lane = lax.iota(jnp.int32, 16)
        svs = [jnp.full((16,), s, jnp.int32) for s in range(8)]
        bufs = (buf0, buf1)
        sems = (sem0, sem1)

        def issue_chunk(q, ci):
            k0, nt = _CHUNKS[ci]
            buf, sem = bufs[ci % 2], sems[ci % 2]
            if k0 >= FULL_TILES:
                pltpu.async_copy(tail_hbm.at[pl.ds(q * 8, 8)], buf.at[0], sem)
            else:
                def issue(kk, c):
                    pltpu.async_copy(
                        tbl_hbm.at[pl.ds(q * 8, 8),
                                   pl.ds((k0 + kk) * 128, 128)],
                        buf.at[kk], sem)
                    return c
                lax.fori_loop(0, nt, issue, 0)

        def wait_chunk(ci):
            k0, nt = _CHUNKS[ci]
            buf, sem = bufs[ci % 2], sems[ci % 2]
            if k0 >= FULL_TILES:
                pltpu.make_async_copy(tail_hbm.at[pl.ds(0, 8)],
                                      buf.at[0], sem).wait()
            else:
                def drain(kk, c):
                    pltpu.make_async_copy(
                        tbl_hbm.at[pl.ds(0, 8), pl.ds(0, 128)],
                        buf.at[kk], sem).wait()
                    return c
                lax.fori_loop(0, nt, drain, 0)

        def compact_chunk(k0, nt):
            c0 = k0 * 128

            def scan(g, cursor):
                ms, encs, prefs, cnts = [], [], [], []
                for u in range(4):
                    gg = g * 4 + u
                    iv = idx_v[pl.ds(gg * 16, 16)]
                    k = lax.shift_right_logical(iv, 7)
                    m = (k >= k0) & (k < k0 + nt)
                    ms.append(m)
                    encs.append(lax.shift_left(lane + gg * 16, 13) | (iv - c0))
                    prefs.append(plsc.cumsum(m.astype(jnp.int32)))
                    cnts.append(plsc.all_reduce_population_count(m))
                for u in range(4):
                    plsc.store_scatter(hits, [cursor + prefs[u] - 1], encs[u],
                                       mask=ms[u])
                    cursor = cursor + cnts[u]
                return cursor

            cursor = lax.fori_loop(0, NVEC // 4, scan,
                                   jnp.zeros((16,), jnp.int32))
            return jnp.max(cursor)

        def gather_chunk(ci, n):
            buf = bufs[ci % 2]
            nn = jnp.full((16,), 1, jnp.int32) * n

            def one(t, c):
                e = hits[pl.ds(t * 16, 16)]
                m2 = (lane + t * 16) < nn
                vl = jnp.where(m2, e & 8191, 0)
                b = jnp.where(m2, lax.shift_right_logical(e, 13), 0)
                kk = lax.shift_right_logical(vl, 7)
                l = vl & 127
                gots = [plsc.load_gather(buf, [kk, svs[s], l], mask=m2)
                        for s in range(8)]
                for s in range(8):
                    plsc.store_scatter(outb, [svs[s], b], gots[s], mask=m2)
                return c
            lax.fori_loop(0, (n + 15) // 16, one, 0)

        def slab_body(j, carry):
            q = wid + NWORK * j

            @pl.when(q < SLABS)
            def _():
                f = q // 8
                pltpu.sync_copy(cat_hbm.at[pl.ds(f * B, B)], idx_v)
                issue_chunk(q, 0)
                for ci, (k0, nt) in enumerate(_CHUNKS):
                    if ci + 1 < len(_CHUNKS):
                        issue_chunk(q, ci + 1)
                    n = compact_chunk(k0, nt)
                    wait_chunk(ci)
                    gather_chunk(ci, n)
                pltpu.sync_copy(outb, out_hbm.at[pl.ds(q * 8, 8)])
            return carry

        lax.fori_loop(0, SLABS_PER_W, slab_body, 0)

    return _sc_gather


BB = 512


def _mlp_body(cont_ref, emb_ref, wc_ref, bc_ref, w1a_ref, w1b_ref, b1_ref,
              w2_ref, b2_ref, w3_ref, b3_ref, wo_ref, bo_ref, out_ref):
    x = jnp.dot(cont_ref[:], wc_ref[:], preferred_element_type=jnp.float32)
    x = x + bc_ref[:]
    a = jnp.dot(x, w1a_ref[:], preferred_element_type=jnp.float32)
    a = a + lax.dot_general(emb_ref[:], w1b_ref[:],
                            dimension_numbers=(((0,), (0,)), ((), ())),
                            preferred_element_type=jnp.float32)
    a = jnp.maximum(a + b1_ref[:], 0.0)
    a = jnp.maximum(jnp.dot(a, w2_ref[:], preferred_element_type=jnp.float32) + b2_ref[:], 0.0)
    a = jnp.maximum(jnp.dot(a, w3_ref[:], preferred_element_type=jnp.float32) + b3_ref[:], 0.0)
    o = jnp.dot(a, wo_ref[:], preferred_element_type=jnp.float32) + bo_ref[:]
    out_ref[:] = jax.nn.sigmoid(o)


def _mlp(cont, embT, W_cont, b_cont, W1a, W1b, b1, W2, b2, W3, b3, Wo, bo):
    grid = (B // BB,)
    full = lambda r, c: pl.BlockSpec((r, c), lambda i: (0, 0))
    return pl.pallas_call(
        _mlp_body,
        grid=grid,
        in_specs=[
            pl.BlockSpec((BB, NF), lambda i: (i, 0)),
            pl.BlockSpec((R, BB), lambda i: (0, i)),
            full(NF, D), full(1, D),
            full(D, H1), full(R, H1), full(1, H1),
            full(H1, H2), full(1, H2),
            full(H2, H3), full(1, H3),
            full(H3, 1), full(1, 1),
        ],
        out_specs=pl.BlockSpec((BB, 1), lambda i: (i, 0)),
        out_shape=jax.ShapeDtypeStruct((B, 1), jnp.float32),
    )(cont, embT, W_cont, b_cont, W1a, W1b, b1, W2, b2, W3, b3, Wo, bo)


def kernel(continuous_features, categorical_features, tables,
           W_cont, b_cont, W1, b1, W2, b2, W3, b3, Wo, bo):
    cat = categorical_features.astype(jnp.int32)
    cat_flat = cat.T.reshape(-1)
    tbl = tables.transpose(0, 2, 1).reshape(R, V)
    tail = jnp.pad(tbl[:, FULL_TILES * 128:], ((0, 0), (0, 128 - TAIL_W)))

    embT = _make_sc_gather()(cat_flat, tbl, tail)

    out = _mlp(continuous_features, embT,
               W_cont, b_cont.reshape(1, D),
               W1[:D], W1[D:], b1.reshape(1, H1),
               W2, b2.reshape(1, H2),
               W3, b3.reshape(1, H3),
               Wo, bo.reshape(1, 1))
    return out


# speedup vs baseline: 1.1065x; 1.0248x over previous
import functools

import jax
import jax.numpy as jnp
from jax import lax
from jax.experimental import pallas as pl
from jax.experimental.pallas import tpu as pltpu
from jax.experimental.pallas import tpu_sc as plsc

B = 4096
NF = 13
NC = 26
V = 100000
D = 64
H1, H2, H3 = 512, 256, 128

R = NC * D
SLABS = R // 8
NWORK = 32
SLABS_PER_W = 7
FULL_TILES = V // 128
TAIL_W = V - FULL_TILES * 128
NT = 43
_CHUNKS = [(i * NT, NT) for i in range(FULL_TILES // NT)]
_REM = FULL_TILES - (FULL_TILES // NT) * NT
_CHUNKS.append((FULL_TILES - _REM, _REM + 1))
NVEC = B // 16


@functools.cache
def _make_sc_gather():
    mesh = plsc.VectorSubcoreMesh(core_axis_name="c", subcore_axis_name="s")

    @functools.partial(
        pl.kernel,
        mesh=mesh,
        compiler_params=pltpu.CompilerParams(needs_layout_passes=False),
        out_type=jax.ShapeDtypeStruct((R, B), jnp.float32),
        scratch_types=[
            pltpu.VMEM((B,), jnp.int32),
            pltpu.VMEM((NT, 8, 128), jnp.float32),
            pltpu.VMEM((NT, 8, 128), jnp.float32),
            pltpu.VMEM((8, B), jnp.float32),
            pltpu.VMEM((B + 32,), jnp.int32),
            pltpu.SemaphoreType.DMA,
            pltpu.SemaphoreType.DMA,
        ],
    )
    def _sc_gather(cat_hbm, tbl_hbm, tail_hbm, out_hbm,
                   idx_v, buf0, buf1, outb, hits, sem0, sem1):
        wid = lax.axis_index("s") * 2 + lax.axis_index("c")

        lane = lax.iota(jnp.int32, 16)
        svs = [jnp.full((16,), s, jnp.int32) for s in range(8)]
        bufs = (buf0, buf1)
        sems = (sem0, sem1)

        def issue_chunk(q, ci):
            k0, nt = _CHUNKS[ci]
            buf, sem = bufs[ci % 2], sems[ci % 2]
            nfull = min(nt, FULL_TILES - k0)

            def issue(kk, c):
                pltpu.async_copy(
                    tbl_hbm.at[pl.ds(q * 8, 8),
                               pl.ds((k0 + kk) * 128, 128)],
                    buf.at[kk], sem)
                return c
            lax.fori_loop(0, nfull, issue, 0)
            if nfull < nt:
                pltpu.async_copy(tail_hbm.at[pl.ds(q * 8, 8)],
                                 buf.at[nfull], sem)

        def wait_chunk(ci):
            k0, nt = _CHUNKS[ci]
            buf, sem = bufs[ci % 2], sems[ci % 2]

            def drain(kk, c):
                pltpu.make_async_copy(
                    tbl_hbm.at[pl.ds(0, 8), pl.ds(0, 128)],
                    buf.at[kk], sem).wait()
                return c
            lax.fori_loop(0, nt, drain, 0)

        def compact_chunk(k0, nt):
            c0 = k0 * 128

            def scan(g, cursor):
                ms, encs, prefs, cnts = [], [], [], []
                for u in range(4):
                    gg = g * 4 + u
                    iv = idx_v[pl.ds(gg * 16, 16)]
                    k = lax.shift_right_logical(iv, 7)
                    m = (k >= k0) & (k < k0 + nt)
                    ms.append(m)
                    encs.append(lax.shift_left(lane + gg * 16, 13) | (iv - c0))
                    prefs.append(plsc.cumsum(m.astype(jnp.int32)))
                    cnts.append(plsc.all_reduce_population_count(m))
                for u in range(4):
                    plsc.store_scatter(hits, [cursor + prefs[u] - 1], encs[u],
                                       mask=ms[u])
                    cursor = cursor + cnts[u]
                return cursor

            cursor = lax.fori_loop(0, NVEC // 4, scan,
                                   jnp.zeros((16,), jnp.int32))
            return jnp.max(cursor)

        def gather_chunk(ci, n):
            buf = bufs[ci % 2]
            nn = jnp.full((16,), 1, jnp.int32) * n

            def one(t, c):
                e = hits[pl.ds(t * 16, 16)]
                m2 = (lane + t * 16) < nn
                vl = jnp.where(m2, e & 8191, 0)
                b = jnp.where(m2, lax.shift_right_logical(e, 13), 0)
                kk = lax.shift_right_logical(vl, 7)
                l = vl & 127
                gots = [plsc.load_gather(buf, [kk, svs[s], l], mask=m2)
                        for s in range(8)]
                for s in range(8):
                    plsc.store_scatter(outb, [svs[s], b], gots[s], mask=m2)
                return c
            lax.fori_loop(0, (n + 15) // 16, one, 0)

        def slab_body(j, carry):
            q = wid + NWORK * j

            @pl.when(q < SLABS)
            def _():
                f = q // 8
                pltpu.sync_copy(cat_hbm.at[pl.ds(f * B, B)], idx_v)
                issue_chunk(q, 0)
                for ci, (k0, nt) in enumerate(_CHUNKS):
                    if ci + 1 < len(_CHUNKS):
                        issue_chunk(q, ci + 1)
                    n = compact_chunk(k0, nt)
                    wait_chunk(ci)
                    gather_chunk(ci, n)
                pltpu.sync_copy(outb, out_hbm.at[pl.ds(q * 8, 8)])
            return carry

        lax.fori_loop(0, SLABS_PER_W, slab_body, 0)

    return _sc_gather


BB = 512


def _mlp_body(cont_ref, emb_ref, wc_ref, bc_ref, w1a_ref, w1b_ref, b1_ref,
              w2_ref, b2_ref, w3_ref, b3_ref, wo_ref, bo_ref, out_ref):
    x = jnp.dot(cont_ref[:], wc_ref[:], preferred_element_type=jnp.float32)
    x = x + bc_ref[:]
    a = jnp.dot(x, w1a_ref[:], preferred_element_type=jnp.float32)
    a = a + lax.dot_general(emb_ref[:], w1b_ref[:],
                            dimension_numbers=(((0,), (0,)), ((), ())),
                            preferred_element_type=jnp.float32)
    a = jnp.maximum(a + b1_ref[:], 0.0)
    a = jnp.maximum(jnp.dot(a, w2_ref[:], preferred_element_type=jnp.float32) + b2_ref[:], 0.0)
    a = jnp.maximum(jnp.dot(a, w3_ref[:], preferred_element_type=jnp.float32) + b3_ref[:], 0.0)
    o = jnp.dot(a, wo_ref[:], preferred_element_type=jnp.float32) + bo_ref[:]
    out_ref[:] = jax.nn.sigmoid(o)


def _mlp(cont, embT, W_cont, b_cont, W1a, W1b, b1, W2, b2, W3, b3, Wo, bo):
    grid = (B // BB,)
    full = lambda r, c: pl.BlockSpec((r, c), lambda i: (0, 0))
    return pl.pallas_call(
        _mlp_body,
        grid=grid,
        in_specs=[
            pl.BlockSpec((BB, NF), lambda i: (i, 0)),
            pl.BlockSpec((R, BB), lambda i: (0, i)),
            full(NF, D), full(1, D),
            full(D, H1), full(R, H1), full(1, H1),
            full(H1, H2), full(1, H2),
            full(H2, H3), full(1, H3),
            full(H3, 1), full(1, 1),
        ],
        out_specs=pl.BlockSpec((BB, 1), lambda i: (i, 0)),
        out_shape=jax.ShapeDtypeStruct((B, 1), jnp.float32),
    )(cont, embT, W_cont, b_cont, W1a, W1b, b1, W2, b2, W3, b3, Wo, bo)


def kernel(continuous_features, categorical_features, tables,
           W_cont, b_cont, W1, b1, W2, b2, W3, b3, Wo, bo):
    cat = categorical_features.astype(jnp.int32)
    cat_flat = cat.T.reshape(-1)
    tbl = tables.transpose(0, 2, 1).reshape(R, V)
    tail = jnp.pad(tbl[:, FULL_TILES * 128:], ((0, 0), (0, 128 - TAIL_W)))

    embT = _make_sc_gather()(cat_flat, tbl, tail)

    out = _mlp(continuous_features, embT,
               W_cont, b_cont.reshape(1, D),
               W1[:D], W1[D:], b1.reshape(1, H1),
               W2, b2.reshape(1, H2),
               W3, b3.reshape(1, H3),
               Wo, bo.reshape(1, 1))
    return out
